# baseline (speedup 1.0000x reference)
import functools

import jax
import jax.numpy as jnp
from jax import lax
from jax.experimental import pallas as pl
from jax.experimental.pallas import tpu as pltpu
from jax.experimental.pallas import tpu_sc as plsc

_B, _T, _D = 4, 4096, 1024
_NW = 32
_TPW = _T // _NW
_R = 16
_NCH = _TPW // _R
_LANES = 16
_VPR = _D // _LANES


def _make_sc_kernel():
    mesh = plsc.VectorSubcoreMesh(core_axis_name="c", subcore_axis_name="s")

    @functools.partial(
        pl.kernel,
        mesh=mesh,
        out_type=jax.ShapeDtypeStruct((_B, _T, _D), jnp.float32),
        scratch_types=[
            pltpu.VMEM((_R, _D), jnp.float32),
            pltpu.VMEM((_B, _R, _D), jnp.float32),
        ],
    )
    def sc_kernel(x_hbm, pe_hbm, out_hbm, pe_v, x_v):
        wid = lax.axis_index("s") * 2 + lax.axis_index("c")
        t0 = wid * _TPW

        def chunk_body(c, carry):
            base = t0 + c * _R
            pltpu.sync_copy(pe_hbm.at[pl.ds(base, _R)], pe_v)
            pltpu.sync_copy(x_hbm.at[:, pl.ds(base, _R)], x_v)

            def row_body(r, carry3):
                for b in range(_B):
                    for j in range(_VPR):
                        v = pe_v[r, pl.ds(j * _LANES, _LANES)]
                        plsc.addupdate(x_v.at[b, r, pl.ds(j * _LANES, _LANES)], v)
                return carry3

            lax.fori_loop(0, _R, row_body, 0)
            pltpu.sync_copy(x_v, out_hbm.at[:, pl.ds(base, _R)])
            return carry

        lax.fori_loop(0, _NCH, chunk_body, 0)

    return sc_kernel


_sc_kernel = _make_sc_kernel()


def kernel(x, positional_embeddings):
    return _sc_kernel(x, positional_embeddings)


# speedup vs baseline: 1.1506x; 1.1506x over previous
import functools

import jax
import jax.numpy as jnp
from jax import lax
from jax.experimental import pallas as pl
from jax.experimental.pallas import tpu as pltpu
from jax.experimental.pallas import tpu_sc as plsc

_B, _T, _D = 4, 4096, 1024
_NW = 32
_TPW = _T // _NW
_R = 4
_NCH = _TPW // _R
_LANES = 16
_VPR = _D // _LANES
_DEPTH = 4


def _make_sc_kernel():
    mesh = plsc.VectorSubcoreMesh(core_axis_name="c", subcore_axis_name="s")

    @functools.partial(
        pl.kernel,
        mesh=mesh,
        out_type=jax.ShapeDtypeStruct((_B, _T, _D), jnp.float32),
        scratch_types=(
            [pltpu.VMEM((_R, _D), jnp.float32)]
            + [pltpu.VMEM((_B, _R, _D), jnp.float32)] * _DEPTH
            + [pltpu.SemaphoreType.DMA] * (2 * _DEPTH)
        ),
    )
    def sc_kernel(x_hbm, pe_hbm, out_hbm, pe_v,
                  x0, x1, x2, x3, g0, g1, g2, g3, s0, s1, s2, s3):
        wid = lax.axis_index("s") * 2 + lax.axis_index("c")
        t0 = wid * _TPW

        bufs = (x0, x1, x2, x3)
        gsems = (g0, g1, g2, g3)
        ssems = (s0, s1, s2, s3)

        def x_src(c):
            return x_hbm.at[:, pl.ds(t0 + c * _R, _R)]

        def out_dst(c):
            return out_hbm.at[:, pl.ds(t0 + c * _R, _R)]

        pltpu.async_copy(x_src(0), bufs[0], gsems[0])
        pltpu.async_copy(x_src(1), bufs[1], gsems[1])

        def lap_body(i, carry):
            c0 = i * _DEPTH
            for p in range(_DEPTH):
                c = c0 + p
                buf, gsem, ssem = bufs[p], gsems[p], ssems[p]
                pltpu.make_async_copy(x_src(c), buf, gsem).wait()
                pltpu.sync_copy(pe_hbm.at[pl.ds(t0 + c * _R, _R)], pe_v)

                def row_body(r, carry3):
                    for b in range(_B):
                        for j in range(_VPR):
                            v = pe_v[r, pl.ds(j * _LANES, _LANES)]
                            plsc.addupdate(
                                buf.at[b, r, pl.ds(j * _LANES, _LANES)], v)
                    return carry3

                lax.fori_loop(0, _R, row_body, 0)
                pltpu.async_copy(buf, out_dst(c), ssem)

                pn = (p + 2) % _DEPTH
                nbuf, ngsem, nssem = bufs[pn], gsems[pn], ssems[pn]

                @pl.when(c + 2 < _NCH)
                def _():
                    @pl.when(c >= 2)
                    def _():
                        pltpu.make_async_copy(
                            nbuf, out_dst(c - 2), nssem).wait()
                    pltpu.async_copy(x_src(c + 2), nbuf, ngsem)

            return carry

        lax.fori_loop(0, _NCH // _DEPTH, lap_body, 0)
        for k in range(_DEPTH):
            c = _NCH - _DEPTH + k
            pltpu.make_async_copy(bufs[c % _DEPTH], out_dst(c),
                                  ssems[c % _DEPTH]).wait()

    return sc_kernel


_sc_kernel = _make_sc_kernel()


def kernel(x, positional_embeddings):
    return _sc_kernel(x, positional_embeddings)


# speedup vs baseline: 2.4193x; 2.1026x over previous
import functools

import jax
import jax.numpy as jnp
from jax import lax
from jax.experimental import pallas as pl
from jax.experimental.pallas import tpu as pltpu
from jax.experimental.pallas import tpu_sc as plsc

_B, _T, _D = 4, 4096, 1024
_NW = 32
_TPW = _T // _NW
_R = 4
_NCH = _TPW // _R
_LANES = 16
_VPR = _D // _LANES
_DEPTH = 4


def _make_sc_kernel():
    mesh = plsc.VectorSubcoreMesh(core_axis_name="c", subcore_axis_name="s")

    @functools.partial(
        pl.kernel,
        mesh=mesh,
        out_type=jax.ShapeDtypeStruct((_B, _T, _D), jnp.float32),
        scratch_types=(
            [pltpu.VMEM((_B, _R, _D), jnp.float32)] * _DEPTH
            + [pltpu.VMEM((_R, _D), jnp.float32)] * _DEPTH
            + [pltpu.SemaphoreType.DMA] * (3 * _DEPTH)
        ),
    )
    def sc_kernel(x_hbm, pe_hbm, out_hbm,
                  x0, x1, x2, x3, p0, p1, p2, p3,
                  g0, g1, g2, g3, s0, s1, s2, s3, q0, q1, q2, q3):
        wid = lax.axis_index("s") * 2 + lax.axis_index("c")
        t0 = wid * _TPW

        xbufs = (x0, x1, x2, x3)
        pbufs = (p0, p1, p2, p3)
        gsems = (g0, g1, g2, g3)
        ssems = (s0, s1, s2, s3)
        qsems = (q0, q1, q2, q3)

        def x_src(c):
            return x_hbm.at[:, pl.ds(t0 + c * _R, _R)]

        def pe_src(c):
            return pe_hbm.at[pl.ds(t0 + c * _R, _R)]

        def out_dst(c):
            return out_hbm.at[:, pl.ds(t0 + c * _R, _R)]

        for c in range(2):
            pltpu.async_copy(x_src(c), xbufs[c], gsems[c])
            pltpu.async_copy(pe_src(c), pbufs[c], qsems[c])

        def lap_body(i, carry):
            c0 = i * _DEPTH
            for p in range(_DEPTH):
                c = c0 + p
                xbuf, pbuf = xbufs[p], pbufs[p]
                pltpu.make_async_copy(x_src(c), xbuf, gsems[p]).wait()
                pltpu.make_async_copy(pe_src(c), pbuf, qsems[p]).wait()

                for b in range(_B):
                    def row_body(r, carry3):
                        @plsc.parallel_loop(0, _VPR, unroll=8)
                        def _(j):
                            v = pbuf[r, pl.ds(j * _LANES, _LANES)]
                            plsc.addupdate(
                                xbuf.at[b, r, pl.ds(j * _LANES, _LANES)], v)
                        return carry3

                    lax.fori_loop(0, _R, row_body, 0)

                pltpu.async_copy(xbuf, out_dst(c), ssems[p])

                pn = (p + 2) % _DEPTH

                @pl.when(c + 2 < _NCH)
                def _():
                    @pl.when(c >= 2)
                    def _():
                        pltpu.make_async_copy(
                            xbufs[pn], out_dst(c - 2), ssems[pn]).wait()
                    pltpu.async_copy(x_src(c + 2), xbufs[pn], gsems[pn])
                    pltpu.async_copy(pe_src(c + 2), pbufs[pn], qsems[pn])

            return carry

        lax.fori_loop(0, _NCH // _DEPTH, lap_body, 0)
        for k in range(_DEPTH):
            c = _NCH - _DEPTH + k
            pltpu.make_async_copy(xbufs[c % _DEPTH], out_dst(c),
                                  ssems[c % _DEPTH]).wait()

    return sc_kernel


_sc_kernel = _make_sc_kernel()


def kernel(x, positional_embeddings):
    return _sc_kernel(x, positional_embeddings)


# speedup vs baseline: 3.4789x; 1.4380x over previous
import jax
import jax.numpy as jnp
from jax.experimental import pallas as pl


def _add_kernel(x_ref, pe_ref, o_ref):
    o_ref[...] = x_ref[...] + pe_ref[...]


def kernel(x, positional_embeddings):
    B, T, D = x.shape
    pe = positional_embeddings
    BT = 512

    grid = (T // BT, B)
    return pl.pallas_call(
        _add_kernel,
        grid=grid,
        in_specs=[
            pl.BlockSpec((1, BT, D), lambda t, b: (b, t, 0)),
            pl.BlockSpec((BT, D), lambda t, b: (t, 0)),
        ],
        out_specs=pl.BlockSpec((1, BT, D), lambda t, b: (b, t, 0)),
        out_shape=jax.ShapeDtypeStruct((B, T, D), x.dtype),
    )(x, pe)


# speedup vs baseline: 3.8570x; 1.1087x over previous
import jax
import jax.numpy as jnp
from jax.experimental import pallas as pl


def _add_kernel(x_ref, pe_ref, o_ref):
    o_ref[...] = x_ref[...] + pe_ref[...]


def kernel(x, positional_embeddings):
    B, T, D = x.shape
    pe = positional_embeddings
    BT = 1024

    grid = (T // BT, B)
    return pl.pallas_call(
        _add_kernel,
        grid=grid,
        in_specs=[
            pl.BlockSpec((1, BT, D), lambda t, b: (b, t, 0)),
            pl.BlockSpec((BT, D), lambda t, b: (t, 0)),
        ],
        out_specs=pl.BlockSpec((1, BT, D), lambda t, b: (b, t, 0)),
        out_shape=jax.ShapeDtypeStruct((B, T, D), x.dtype),
    )(x, pe)


# speedup vs baseline: 4.1075x; 1.0649x over previous
import jax
import jax.numpy as jnp
from jax.experimental import pallas as pl


def _add_kernel(x_ref, pe_ref, o_ref):
    o_ref[...] = x_ref[...] + pe_ref[...]


def kernel(x, positional_embeddings):
    B, T, D = x.shape
    pe = positional_embeddings
    BT = 2048

    grid = (T // BT, B)
    return pl.pallas_call(
        _add_kernel,
        grid=grid,
        in_specs=[
            pl.BlockSpec((1, BT, D), lambda t, b: (b, t, 0)),
            pl.BlockSpec((BT, D), lambda t, b: (t, 0)),
        ],
        out_specs=pl.BlockSpec((1, BT, D), lambda t, b: (b, t, 0)),
        out_shape=jax.ShapeDtypeStruct((B, T, D), x.dtype),
    )(x, pe)
